# Initial kernel scaffold; baseline (speedup 1.0000x reference)
import jax
import jax.numpy as jnp
from jax.experimental import pallas as pl


def kernel(x, positions, c, Wq, bq, Wk, bk, Wv, bv, Wo, bo, W1, b1, W2, b2, ln1_scale, ln1_bias, ln2_scale, ln2_bias, log_tau, attn_scale):
    raise NotImplementedError("write your pallas kernel here")


# speedup vs baseline: 13.0705x; 13.0705x over previous
import math

import jax
import jax.numpy as jnp
from jax import lax
from jax.experimental import pallas as pl
from jax.experimental.pallas import tpu as pltpu

N = 2048
DIM = 768
NHEADS = 12
HD = DIM // NHEADS
KNN = 16
PDIM = 16
FF = 4 * DIM
BLK = 256
GRID = N // BLK


def _ln(x, scale, bias):
    m = jnp.mean(x, axis=-1, keepdims=True)
    xc = x - m
    v = jnp.mean(xc * xc, axis=-1, keepdims=True)
    return xc / jnp.sqrt(v + 1e-6) * scale + bias


def _qkv_body(x_ref, w_ref, b_ref, s_ref, t_ref, o_ref):
    xn = _ln(x_ref[...], s_ref[...], t_ref[...])
    o_ref[...] = (
        jnp.dot(xn, w_ref[...], preferred_element_type=jnp.float32) + b_ref[...]
    )


def _topk_body(pb_ref, pf_ref, c_ref, idx_ref, dist_ref):
    c = c_ref[0, 0]
    pb = pb_ref[...]
    pf = pf_ref[...]
    nb = jnp.sum(pb * pb, axis=1, keepdims=True)
    ones = jnp.ones((1, PDIM), jnp.float32)
    nf = lax.dot_general(
        ones, pf * pf, (((1,), (1,)), ((), ())),
        precision=lax.Precision.HIGHEST,
        preferred_element_type=jnp.float32)
    g = lax.dot_general(
        pb, pf, (((1,), (1,)), ((), ())),
        precision=lax.Precision.HIGHEST,
        preferred_element_type=jnp.float32)
    diff = jnp.maximum(nb + nf - 2.0 * g, 0.0)
    den = jnp.maximum((1.0 - c * nb) * (1.0 - c * nf), 1e-8)
    arg = 1.0 + 2.0 * c * diff / den

    iota = lax.broadcasted_iota(jnp.int32, (BLK, N), 1)
    iota16 = lax.broadcasted_iota(jnp.int32, (BLK, KNN), 1)
    work = arg
    idx_acc = jnp.zeros((BLK, KNN), jnp.int32)
    arg_acc = jnp.zeros((BLK, KNN), jnp.float32)
    for t in range(KNN):
        m = jnp.min(work, axis=1, keepdims=True)
        j = jnp.min(jnp.where(work == m, iota, N), axis=1, keepdims=True)
        idx_acc = jnp.where(iota16 == t, j, idx_acc)
        arg_acc = jnp.where(iota16 == t, m, arg_acc)
        work = jnp.where(iota == j, jnp.float32(jnp.inf), work)
    idx_ref[...] = idx_acc
    s = jnp.maximum(arg_acc, 1.0 + 1e-7)
    dist_ref[...] = (
        jnp.log(s + jnp.sqrt((s - 1.0) * (s + 1.0))) / jnp.sqrt(c)
    )


def _attn_body(q_ref, k_ref, v_ref, idx_ref, dist_ref, x_ref, wo_ref,
               bo_ref, lt_ref, as_ref, o_ref):
    inv_tau = 1.0 / jnp.maximum(jnp.exp(lt_ref[0, 0]), 1e-8)
    a_scale = as_ref[0, 0]
    idx = idx_ref[...]
    dist = dist_ref[...]
    iota = lax.broadcasted_iota(jnp.int32, (BLK, N), 1)
    mask = jnp.zeros((BLK, N), jnp.bool_)
    geo = jnp.zeros((BLK, N), jnp.float32)
    for t in range(KNN):
        hit = iota == idx[:, t:t + 1]
        mask = jnp.logical_or(mask, hit)
        geo = jnp.where(hit, -dist[:, t:t + 1] * inv_tau, geo)

    inv_sqrt_hd = 1.0 / math.sqrt(HD)
    acc = x_ref[...] + bo_ref[...]
    for h in range(NHEADS):
        qh = q_ref[:, h * HD:(h + 1) * HD]
        kh = k_ref[:, h * HD:(h + 1) * HD]
        vh = v_ref[:, h * HD:(h + 1) * HD]
        s = lax.dot_general(
            qh, kh, (((1,), (1,)), ((), ())),
            preferred_element_type=jnp.float32) * inv_sqrt_hd
        sc = a_scale * jnp.tanh(s + geo)
        w = jnp.where(mask, jnp.exp(sc), 0.0)
        w = w / jnp.sum(w, axis=1, keepdims=True)
        oh = jnp.dot(w, vh, preferred_element_type=jnp.float32)
        acc = acc + jnp.dot(
            oh, wo_ref[h * HD:(h + 1) * HD, :],
            preferred_element_type=jnp.float32)
    o_ref[...] = acc


def _mlp_body(x_ref, w1_ref, b1_ref, w2_ref, b2_ref, s_ref, t_ref, o_ref):
    xo = x_ref[...]
    h = _ln(xo, s_ref[...], t_ref[...])
    a = jax.nn.gelu(
        jnp.dot(h, w1_ref[...], preferred_element_type=jnp.float32)
        + b1_ref[...])
    o_ref[...] = xo + (
        jnp.dot(a, w2_ref[...], preferred_element_type=jnp.float32)
        + b2_ref[...])


def kernel(x, positions, c, Wq, bq, Wk, bk, Wv, bv, Wo, bo, W1, b1, W2, b2,
           ln1_scale, ln1_bias, ln2_scale, ln2_bias, log_tau, attn_scale):
    x0 = x[0]
    pos = positions[0]
    Wqkv = jnp.concatenate([Wq, Wk, Wv], axis=1)
    bqkv = jnp.concatenate([bq, bk, bv])[None, :]
    c2 = jnp.reshape(c, (1, 1)).astype(jnp.float32)
    lt2 = jnp.reshape(log_tau, (1, 1)).astype(jnp.float32)
    as2 = jnp.reshape(attn_scale, (1, 1)).astype(jnp.float32)

    qkv = pl.pallas_call(
        _qkv_body,
        grid=(GRID,),
        in_specs=[
            pl.BlockSpec((BLK, DIM), lambda i: (i, 0)),
            pl.BlockSpec((DIM, 3 * DIM), lambda i: (0, 0)),
            pl.BlockSpec((1, 3 * DIM), lambda i: (0, 0)),
            pl.BlockSpec((1, DIM), lambda i: (0, 0)),
            pl.BlockSpec((1, DIM), lambda i: (0, 0)),
        ],
        out_specs=pl.BlockSpec((BLK, 3 * DIM), lambda i: (i, 0)),
        out_shape=jax.ShapeDtypeStruct((N, 3 * DIM), jnp.float32),
    )(x0, Wqkv, bqkv, ln1_scale[None, :], ln1_bias[None, :])

    idx, dist = pl.pallas_call(
        _topk_body,
        grid=(GRID,),
        in_specs=[
            pl.BlockSpec((BLK, PDIM), lambda i: (i, 0)),
            pl.BlockSpec((N, PDIM), lambda i: (0, 0)),
            pl.BlockSpec((1, 1), lambda i: (0, 0)),
        ],
        out_specs=[
            pl.BlockSpec((BLK, KNN), lambda i: (i, 0)),
            pl.BlockSpec((BLK, KNN), lambda i: (i, 0)),
        ],
        out_shape=[
            jax.ShapeDtypeStruct((N, KNN), jnp.int32),
            jax.ShapeDtypeStruct((N, KNN), jnp.float32),
        ],
    )(pos, pos, c2)

    x_out = pl.pallas_call(
        _attn_body,
        grid=(GRID,),
        in_specs=[
            pl.BlockSpec((BLK, DIM), lambda i: (i, 0)),
            pl.BlockSpec((N, DIM), lambda i: (0, 1)),
            pl.BlockSpec((N, DIM), lambda i: (0, 2)),
            pl.BlockSpec((BLK, KNN), lambda i: (i, 0)),
            pl.BlockSpec((BLK, KNN), lambda i: (i, 0)),
            pl.BlockSpec((BLK, DIM), lambda i: (i, 0)),
            pl.BlockSpec((DIM, DIM), lambda i: (0, 0)),
            pl.BlockSpec((1, DIM), lambda i: (0, 0)),
            pl.BlockSpec((1, 1), lambda i: (0, 0)),
            pl.BlockSpec((1, 1), lambda i: (0, 0)),
        ],
        out_specs=pl.BlockSpec((BLK, DIM), lambda i: (i, 0)),
        out_shape=jax.ShapeDtypeStruct((N, DIM), jnp.float32),
    )(qkv, qkv, qkv, idx, dist, x0, Wo, bo[None, :], lt2, as2)

    y = pl.pallas_call(
        _mlp_body,
        grid=(GRID,),
        in_specs=[
            pl.BlockSpec((BLK, DIM), lambda i: (i, 0)),
            pl.BlockSpec((DIM, FF), lambda i: (0, 0)),
            pl.BlockSpec((1, FF), lambda i: (0, 0)),
            pl.BlockSpec((FF, DIM), lambda i: (0, 0)),
            pl.BlockSpec((1, DIM), lambda i: (0, 0)),
            pl.BlockSpec((1, DIM), lambda i: (0, 0)),
            pl.BlockSpec((1, DIM), lambda i: (0, 0)),
        ],
        out_specs=pl.BlockSpec((BLK, DIM), lambda i: (i, 0)),
        out_shape=jax.ShapeDtypeStruct((N, DIM), jnp.float32),
    )(x_out, W1, b1[None, :], W2, b2[None, :],
      ln2_scale[None, :], ln2_bias[None, :])

    return y[None]


# speedup vs baseline: 14.0087x; 1.0718x over previous
import math

import jax
import jax.numpy as jnp
from jax import lax
from jax.experimental import pallas as pl
from jax.experimental.pallas import tpu as pltpu

N = 2048
DIM = 768
NHEADS = 12
HD = DIM // NHEADS
KNN = 16
PDIM = 16
FF = 4 * DIM
BLK = 256
GRID = N // BLK


def _ln(x, scale, bias):
    m = jnp.mean(x, axis=-1, keepdims=True)
    xc = x - m
    v = jnp.mean(xc * xc, axis=-1, keepdims=True)
    return xc / jnp.sqrt(v + 1e-6) * scale + bias


def _qkv_body(x_ref, w_ref, b_ref, s_ref, t_ref, o_ref):
    xn = _ln(x_ref[...], s_ref[...], t_ref[...])
    o_ref[...] = (
        jnp.dot(xn, w_ref[...], preferred_element_type=jnp.float32) + b_ref[...]
    )


def _topk_body(pb_ref, pf_ref, c_ref, idx_ref, dist_ref):
    c = c_ref[0, 0]
    pb = pb_ref[...]
    pf = pf_ref[...]
    nb = jnp.sum(pb * pb, axis=1, keepdims=True)
    ones = jnp.ones((1, PDIM), jnp.float32)
    nf = lax.dot_general(
        ones, pf * pf, (((1,), (1,)), ((), ())),
        precision=lax.Precision.HIGHEST,
        preferred_element_type=jnp.float32)
    g = lax.dot_general(
        pb, pf, (((1,), (1,)), ((), ())),
        precision=lax.Precision.HIGHEST,
        preferred_element_type=jnp.float32)
    diff = jnp.maximum(nb + nf - 2.0 * g, 0.0)
    den = jnp.maximum((1.0 - c * nb) * (1.0 - c * nf), 1e-8)
    arg = 1.0 + 2.0 * c * diff / den

    iota = lax.broadcasted_iota(jnp.int32, (BLK, N), 1)
    iota16 = lax.broadcasted_iota(jnp.int32, (BLK, KNN), 1)
    work = arg
    idx_acc = jnp.zeros((BLK, KNN), jnp.int32)
    arg_acc = jnp.zeros((BLK, KNN), jnp.float32)
    for t in range(KNN):
        m = jnp.min(work, axis=1, keepdims=True)
        j = jnp.min(jnp.where(work == m, iota, N), axis=1, keepdims=True)
        idx_acc = jnp.where(iota16 == t, j, idx_acc)
        arg_acc = jnp.where(iota16 == t, m, arg_acc)
        work = jnp.where(iota == j, jnp.float32(jnp.inf), work)
    idx_ref[...] = idx_acc
    s = jnp.maximum(arg_acc, 1.0 + 1e-7)
    dist_ref[...] = (
        jnp.log(s + jnp.sqrt((s - 1.0) * (s + 1.0))) / jnp.sqrt(c)
    )


def _attn_body(q_ref, k_ref, v_ref, idx_ref, dist_ref, x_ref, wo_ref,
               bo_ref, lt_ref, as_ref, o_ref):
    inv_tau = 1.0 / jnp.maximum(jnp.exp(lt_ref[0, 0]), 1e-8)
    a_scale = as_ref[0, 0]
    idx = idx_ref[...]
    dist = dist_ref[...]
    iota = lax.broadcasted_iota(jnp.int32, (BLK, N), 1)
    geo = jnp.zeros((BLK, N), jnp.float32)
    for t in range(KNN):
        geo = jnp.where(iota == idx[:, t:t + 1],
                        -dist[:, t:t + 1] * inv_tau, geo)
    mask = geo < 0.0

    inv_sqrt_hd = 1.0 / math.sqrt(HD)
    ones8 = jnp.ones((N, 8), jnp.float32)
    acc = x_ref[...] + bo_ref[...]
    for h in range(NHEADS):
        qh = q_ref[:, h * HD:(h + 1) * HD]
        kh = k_ref[:, h * HD:(h + 1) * HD]
        vh = v_ref[:, h * HD:(h + 1) * HD]
        s = lax.dot_general(
            qh, kh, (((1,), (1,)), ((), ())),
            preferred_element_type=jnp.float32) * inv_sqrt_hd
        sc = a_scale * jnp.tanh(s + geo)
        w = jnp.where(mask, jnp.exp(sc), 0.0)
        denom = jnp.dot(w, ones8, preferred_element_type=jnp.float32)
        oh = jnp.dot(w, vh, preferred_element_type=jnp.float32)
        oh = oh * (1.0 / denom[:, 0:1])
        acc = acc + jnp.dot(
            oh, wo_ref[h * HD:(h + 1) * HD, :],
            preferred_element_type=jnp.float32)
    o_ref[...] = acc


def _mlp_body(x_ref, w1_ref, b1_ref, w2_ref, b2_ref, s_ref, t_ref, o_ref):
    xo = x_ref[...]
    h = _ln(xo, s_ref[...], t_ref[...])
    a = jax.nn.gelu(
        jnp.dot(h, w1_ref[...], preferred_element_type=jnp.float32)
        + b1_ref[...])
    o_ref[...] = xo + (
        jnp.dot(a, w2_ref[...], preferred_element_type=jnp.float32)
        + b2_ref[...])


def kernel(x, positions, c, Wq, bq, Wk, bk, Wv, bv, Wo, bo, W1, b1, W2, b2,
           ln1_scale, ln1_bias, ln2_scale, ln2_bias, log_tau, attn_scale):
    x0 = x[0]
    pos = positions[0]
    Wqkv = jnp.concatenate([Wq, Wk, Wv], axis=1)
    bqkv = jnp.concatenate([bq, bk, bv])[None, :]
    c2 = jnp.reshape(c, (1, 1)).astype(jnp.float32)
    lt2 = jnp.reshape(log_tau, (1, 1)).astype(jnp.float32)
    as2 = jnp.reshape(attn_scale, (1, 1)).astype(jnp.float32)

    qkv = pl.pallas_call(
        _qkv_body,
        grid=(GRID,),
        in_specs=[
            pl.BlockSpec((BLK, DIM), lambda i: (i, 0)),
            pl.BlockSpec((DIM, 3 * DIM), lambda i: (0, 0)),
            pl.BlockSpec((1, 3 * DIM), lambda i: (0, 0)),
            pl.BlockSpec((1, DIM), lambda i: (0, 0)),
            pl.BlockSpec((1, DIM), lambda i: (0, 0)),
        ],
        out_specs=pl.BlockSpec((BLK, 3 * DIM), lambda i: (i, 0)),
        out_shape=jax.ShapeDtypeStruct((N, 3 * DIM), jnp.float32),
    )(x0, Wqkv, bqkv, ln1_scale[None, :], ln1_bias[None, :])

    idx, dist = pl.pallas_call(
        _topk_body,
        grid=(GRID,),
        in_specs=[
            pl.BlockSpec((BLK, PDIM), lambda i: (i, 0)),
            pl.BlockSpec((N, PDIM), lambda i: (0, 0)),
            pl.BlockSpec((1, 1), lambda i: (0, 0)),
        ],
        out_specs=[
            pl.BlockSpec((BLK, KNN), lambda i: (i, 0)),
            pl.BlockSpec((BLK, KNN), lambda i: (i, 0)),
        ],
        out_shape=[
            jax.ShapeDtypeStruct((N, KNN), jnp.int32),
            jax.ShapeDtypeStruct((N, KNN), jnp.float32),
        ],
    )(pos, pos, c2)

    x_out = pl.pallas_call(
        _attn_body,
        grid=(GRID,),
        in_specs=[
            pl.BlockSpec((BLK, DIM), lambda i: (i, 0)),
            pl.BlockSpec((N, DIM), lambda i: (0, 1)),
            pl.BlockSpec((N, DIM), lambda i: (0, 2)),
            pl.BlockSpec((BLK, KNN), lambda i: (i, 0)),
            pl.BlockSpec((BLK, KNN), lambda i: (i, 0)),
            pl.BlockSpec((BLK, DIM), lambda i: (i, 0)),
            pl.BlockSpec((DIM, DIM), lambda i: (0, 0)),
            pl.BlockSpec((1, DIM), lambda i: (0, 0)),
            pl.BlockSpec((1, 1), lambda i: (0, 0)),
            pl.BlockSpec((1, 1), lambda i: (0, 0)),
        ],
        out_specs=pl.BlockSpec((BLK, DIM), lambda i: (i, 0)),
        out_shape=jax.ShapeDtypeStruct((N, DIM), jnp.float32),
    )(qkv, qkv, qkv, idx, dist, x0, Wo, bo[None, :], lt2, as2)

    y = pl.pallas_call(
        _mlp_body,
        grid=(GRID,),
        in_specs=[
            pl.BlockSpec((BLK, DIM), lambda i: (i, 0)),
            pl.BlockSpec((DIM, FF), lambda i: (0, 0)),
            pl.BlockSpec((1, FF), lambda i: (0, 0)),
            pl.BlockSpec((FF, DIM), lambda i: (0, 0)),
            pl.BlockSpec((1, DIM), lambda i: (0, 0)),
            pl.BlockSpec((1, DIM), lambda i: (0, 0)),
            pl.BlockSpec((1, DIM), lambda i: (0, 0)),
        ],
        out_specs=pl.BlockSpec((BLK, DIM), lambda i: (i, 0)),
        out_shape=jax.ShapeDtypeStruct((N, DIM), jnp.float32),
    )(x_out, W1, b1[None, :], W2, b2[None, :],
      ln2_scale[None, :], ln2_bias[None, :])

    return y[None]


# speedup vs baseline: 14.3628x; 1.0253x over previous
import math

import jax
import jax.numpy as jnp
from jax import lax
from jax.experimental import pallas as pl
from jax.experimental.pallas import tpu as pltpu

N = 2048
DIM = 768
NHEADS = 12
HD = DIM // NHEADS
KNN = 16
PDIM = 16
FF = 4 * DIM
BLK = 256
GRID = N // BLK


def _ln(x, scale, bias):
    m = jnp.mean(x, axis=-1, keepdims=True)
    xc = x - m
    v = jnp.mean(xc * xc, axis=-1, keepdims=True)
    return xc / jnp.sqrt(v + 1e-6) * scale + bias


def _qkv_body(x_ref, w_ref, b_ref, s_ref, t_ref, o_ref):
    xn = _ln(x_ref[...], s_ref[...], t_ref[...])
    o_ref[...] = (
        jnp.dot(xn, w_ref[...], preferred_element_type=jnp.float32) + b_ref[...]
    )


def _fused_body(pb_ref, pf_ref, q_ref, k_ref, v_ref, x_ref, wo_ref, bo_ref,
                w1_ref, b1_ref, w2_ref, b2_ref, s2_ref, t2_ref,
                c_ref, lt_ref, as_ref, o_ref):
    c = c_ref[0, 0]
    inv_tau = 1.0 / jnp.maximum(jnp.exp(lt_ref[0, 0]), 1e-8)
    a_scale = as_ref[0, 0]
    inv_sqrt_c = 1.0 / jnp.sqrt(c)

    pb = pb_ref[...]
    pf = pf_ref[...]
    nb = jnp.sum(pb * pb, axis=1, keepdims=True)
    ones_p = jnp.ones((1, PDIM), jnp.float32)
    nf = lax.dot_general(
        ones_p, pf * pf, (((1,), (1,)), ((), ())),
        precision=lax.Precision.HIGHEST,
        preferred_element_type=jnp.float32)
    g = lax.dot_general(
        pb, pf, (((1,), (1,)), ((), ())),
        precision=lax.Precision.HIGHEST,
        preferred_element_type=jnp.float32)
    diff = jnp.maximum(nb + nf - 2.0 * g, 0.0)
    den = jnp.maximum((1.0 - c * nb) * (1.0 - c * nf), 1e-8)
    arg = 1.0 + 2.0 * c * diff / den

    iota = lax.broadcasted_iota(jnp.int32, (BLK, N), 1)
    work = arg
    geo = jnp.zeros((BLK, N), jnp.float32)
    for _ in range(KNN):
        m = jnp.min(work, axis=1, keepdims=True)
        j = jnp.min(jnp.where(work == m, iota, N), axis=1, keepdims=True)
        s = jnp.maximum(m, 1.0 + 1e-7)
        d = jnp.log(s + jnp.sqrt((s - 1.0) * (s + 1.0))) * inv_sqrt_c
        hit = iota == j
        work = jnp.where(hit, jnp.float32(jnp.inf), work)
        geo = jnp.where(hit, -d * inv_tau, geo)
    mask = geo < 0.0

    inv_sqrt_hd = 1.0 / math.sqrt(HD)
    ones8 = jnp.ones((N, 8), jnp.float32)
    acc = x_ref[...] + bo_ref[...]
    for h in range(NHEADS):
        qh = q_ref[:, h * HD:(h + 1) * HD]
        kh = k_ref[:, h * HD:(h + 1) * HD]
        vh = v_ref[:, h * HD:(h + 1) * HD]
        sco = lax.dot_general(
            qh, kh, (((1,), (1,)), ((), ())),
            preferred_element_type=jnp.float32) * inv_sqrt_hd
        sc = a_scale * jnp.tanh(sco + geo)
        w = jnp.where(mask, jnp.exp(sc), 0.0)
        denom = jnp.dot(w, ones8, preferred_element_type=jnp.float32)
        oh = jnp.dot(w, vh, preferred_element_type=jnp.float32)
        oh = oh * (1.0 / denom[:, 0:1])
        acc = acc + jnp.dot(
            oh, wo_ref[h * HD:(h + 1) * HD, :],
            preferred_element_type=jnp.float32)

    hh = _ln(acc, s2_ref[...], t2_ref[...])
    a = jax.nn.gelu(
        jnp.dot(hh, w1_ref[...], preferred_element_type=jnp.float32)
        + b1_ref[...])
    o_ref[...] = acc + (
        jnp.dot(a, w2_ref[...], preferred_element_type=jnp.float32)
        + b2_ref[...])


def kernel(x, positions, c, Wq, bq, Wk, bk, Wv, bv, Wo, bo, W1, b1, W2, b2,
           ln1_scale, ln1_bias, ln2_scale, ln2_bias, log_tau, attn_scale):
    x0 = x[0]
    pos = positions[0]
    Wqkv = jnp.concatenate([Wq, Wk, Wv], axis=1)
    bqkv = jnp.concatenate([bq, bk, bv])[None, :]
    c2 = jnp.reshape(c, (1, 1)).astype(jnp.float32)
    lt2 = jnp.reshape(log_tau, (1, 1)).astype(jnp.float32)
    as2 = jnp.reshape(attn_scale, (1, 1)).astype(jnp.float32)

    qkv = pl.pallas_call(
        _qkv_body,
        grid=(GRID,),
        in_specs=[
            pl.BlockSpec((BLK, DIM), lambda i: (i, 0)),
            pl.BlockSpec((DIM, 3 * DIM), lambda i: (0, 0)),
            pl.BlockSpec((1, 3 * DIM), lambda i: (0, 0)),
            pl.BlockSpec((1, DIM), lambda i: (0, 0)),
            pl.BlockSpec((1, DIM), lambda i: (0, 0)),
        ],
        out_specs=pl.BlockSpec((BLK, 3 * DIM), lambda i: (i, 0)),
        out_shape=jax.ShapeDtypeStruct((N, 3 * DIM), jnp.float32),
    )(x0, Wqkv, bqkv, ln1_scale[None, :], ln1_bias[None, :])

    y = pl.pallas_call(
        _fused_body,
        grid=(GRID,),
        in_specs=[
            pl.BlockSpec((BLK, PDIM), lambda i: (i, 0)),
            pl.BlockSpec((N, PDIM), lambda i: (0, 0)),
            pl.BlockSpec((BLK, DIM), lambda i: (i, 0)),
            pl.BlockSpec((N, DIM), lambda i: (0, 1)),
            pl.BlockSpec((N, DIM), lambda i: (0, 2)),
            pl.BlockSpec((BLK, DIM), lambda i: (i, 0)),
            pl.BlockSpec((DIM, DIM), lambda i: (0, 0)),
            pl.BlockSpec((1, DIM), lambda i: (0, 0)),
            pl.BlockSpec((DIM, FF), lambda i: (0, 0)),
            pl.BlockSpec((1, FF), lambda i: (0, 0)),
            pl.BlockSpec((FF, DIM), lambda i: (0, 0)),
            pl.BlockSpec((1, DIM), lambda i: (0, 0)),
            pl.BlockSpec((1, DIM), lambda i: (0, 0)),
            pl.BlockSpec((1, DIM), lambda i: (0, 0)),
            pl.BlockSpec((1, 1), lambda i: (0, 0)),
            pl.BlockSpec((1, 1), lambda i: (0, 0)),
            pl.BlockSpec((1, 1), lambda i: (0, 0)),
        ],
        out_specs=pl.BlockSpec((BLK, DIM), lambda i: (i, 0)),
        out_shape=jax.ShapeDtypeStruct((N, DIM), jnp.float32),
    )(pos, pos, qkv, qkv, qkv, x0, Wo, bo[None, :], W1, b1[None, :],
      W2, b2[None, :], ln2_scale[None, :], ln2_bias[None, :], c2, lt2, as2)

    return y[None]
